# baseline (speedup 1.0000x reference)
import functools

import jax
import jax.numpy as jnp
from jax import lax
from jax.experimental import pallas as pl
from jax.experimental.pallas import tpu as pltpu
from jax.experimental.pallas import tpu_sc as plsc

N = 100000
NC, NS, L = 2, 16, 16
NW = NC * NS
SEG = 208
WPS = SEG // L
NSEG = NW * L
NP = NSEG * SEG
NWIN = NSEG * WPS
CHUNK = SEG * L
NU = 4096
UPW = NU // NW
UVR = UPW // L
SEG_STEPS = 9
WIN_STEPS = 4
POS_STEPS = 4

_mesh = plsc.VectorSubcoreMesh(
    core_axis_name="c", subcore_axis_name="s", num_cores=NC, num_subcores=NS
)
_params = pltpu.CompilerParams(
    needs_layout_passes=False, use_tc_tiling_on_sc=False
)


def _wid():
    return lax.axis_index("s") * NC + lax.axis_index("c")


def _scan_body(x_ref, o_ref):
    x = x_ref[...]
    sh = 1
    while sh < SEG:
        top = jnp.zeros((sh, NSEG), jnp.float32)
        x = x + jnp.concatenate([top, x[: SEG - sh]], axis=0)
        sh *= 2
    o_ref[...] = x


_tc_scan = pl.pallas_call(
    _scan_body, out_shape=jax.ShapeDtypeStruct((SEG, NSEG), jnp.float32)
)


@functools.partial(
    pl.kernel,
    out_type=jax.ShapeDtypeStruct((NU,), jnp.int32),
    mesh=_mesh,
    compiler_params=_params,
    scratch_types=[
        pltpu.VMEM((NSEG,), jnp.float32),
        pltpu.VMEM((NSEG,), jnp.float32),
        pltpu.VMEM((NSEG,), jnp.float32),
        pltpu.VMEM((NWIN,), jnp.float32),
        pltpu.VMEM((UPW,), jnp.float32),
        pltpu.VMEM((2, UPW // 2), jnp.int32),
        pltpu.VMEM((2, UPW // 2, L), jnp.float32),
        pltpu.VMEM((UPW,), jnp.int32),
        pltpu.SemaphoreType.DMA,
        pltpu.SemaphoreType.DMA,
    ],
)
def _search(lcdfw_hbm, win_hbm, segtot_hbm, u_hbm, out_hbm,
            sb_v, gseg_v, gexc_v, win_v, u_v, widx_v, rows_v, out_v,
            sem, sem2):
    w = _wid()
    cps = (
        pltpu.async_copy(segtot_hbm, sb_v, sem),
        pltpu.async_copy(win_hbm, win_v, sem),
        pltpu.async_copy(u_hbm.at[pl.ds(w * UPW, UPW)], u_v, sem),
    )
    for cp in cps:
        cp.wait()

    carry = jnp.float32(0.0)
    for r in range(NSEG // L):
        v = sb_v[pl.ds(r * L, L)]
        cinc = plsc.cumsum(v) + carry
        gseg_v[pl.ds(r * L, L)] = cinc
        gexc_v[pl.ds(r * L, L)] = cinc - v
        carry = cinc[L - 1]
    total = carry

    ii = lax.iota(jnp.int32, L)
    HB = UVR // 2

    def _levels12(jset):
        ts = {j: u_v[pl.ds(j * L, L)] * total for j in jset}
        los = {j: jnp.zeros((L,), jnp.int32) for j in jset}
        his = {j: jnp.full((L,), NSEG - 1, jnp.int32) for j in jset}
        for _ in range(SEG_STEPS):
            for j in jset:
                mid = lax.shift_right_logical(los[j] + his[j], 1)
                g = plsc.load_gather(gseg_v, [mid])
                left = ts[j] <= g
                los[j] = jnp.where(left, los[j], mid + 1)
                his[j] = jnp.where(left, mid, his[j])
        tts, bases = {}, {}
        for j in jset:
            s = los[j]
            tts[j] = ts[j] - plsc.load_gather(gexc_v, [s])
            bases[j] = s * WPS
            los[j] = jnp.zeros((L,), jnp.int32)
            his[j] = jnp.full((L,), WPS - 1, jnp.int32)
        for _ in range(WIN_STEPS):
            for j in jset:
                mid = lax.shift_right_logical(los[j] + his[j], 1)
                g = plsc.load_gather(win_v, [bases[j] + mid])
                left = tts[j] <= g
                los[j] = jnp.where(left, los[j], mid + 1)
                his[j] = jnp.where(left, mid, his[j])
        widxs = {}
        for j in jset:
            widxs[j] = bases[j] + los[j]
            widx_v[j // HB, pl.ds((j % HB) * L, L)] = widxs[j]
        return tts, widxs

    def _level3(jset, tts, widxs, h):
        los = {j: jnp.zeros((L,), jnp.int32) for j in jset}
        his = {j: jnp.full((L,), L - 1, jnp.int32) for j in jset}
        hv = jnp.full((L,), h, jnp.int32)
        for _ in range(POS_STEPS):
            for j in jset:
                mid = lax.shift_right_logical(los[j] + his[j], 1)
                c = plsc.load_gather(
                    rows_v, [hv, ii + (j % HB) * L, mid]
                )
                left = tts[j] <= c
                los[j] = jnp.where(left, los[j], mid + 1)
                his[j] = jnp.where(left, mid, his[j])
        for j in jset:
            out_v[pl.ds(j * L, L)] = jnp.minimum(widxs[j] * L + los[j], N)

    set_a = range(0, HB)
    set_b = range(HB, UVR)
    tts_a, widxs_a = _levels12(set_a)
    cp_a = pltpu.async_copy(lcdfw_hbm.at[widx_v.at[0]], rows_v.at[0], sem)
    tts_b, widxs_b = _levels12(set_b)
    cp_b = pltpu.async_copy(lcdfw_hbm.at[widx_v.at[1]], rows_v.at[1], sem2)
    cp_a.wait()
    _level3(set_a, tts_a, widxs_a, 0)
    cp_b.wait()
    _level3(set_b, tts_b, widxs_b, 1)
    pltpu.sync_copy(out_v, out_hbm.at[pl.ds(w * UPW, UPW)])


def kernel(probs, x):
    dims = tuple(x.shape[:-1]) + (1,)
    n = 1
    for d in dims:
        n *= d
    assert n == NU and probs.shape == (N,)
    pp = jnp.concatenate(
        [probs.astype(jnp.float32), jnp.zeros((NP - N,), jnp.float32)]
    ).reshape(NSEG, SEG)
    u = jax.random.uniform(jax.random.key(42), (n,), dtype=jnp.float32)
    lcdf = jnp.transpose(_tc_scan(jnp.transpose(pp)))
    win = lcdf[:, L - 1 :: L].reshape(NWIN)
    segtot = lcdf[:, SEG - 1]
    samples = _search(lcdf.reshape(NWIN, L), win, segtot, u)
    return samples.reshape(dims)


# speedup vs baseline: 1.0015x; 1.0015x over previous
import functools

import jax
import jax.numpy as jnp
from jax import lax
from jax.experimental import pallas as pl
from jax.experimental.pallas import tpu as pltpu
from jax.experimental.pallas import tpu_sc as plsc

N = 100000
NC, NS, L = 2, 16, 16
NW = NC * NS
SEG = 208
WPS = SEG // L
NSEG = NW * L
NP = NSEG * SEG
NWIN = NSEG * WPS
CHUNK = SEG * L
NU = 4096
UPW = NU // NW
UVR = UPW // L
SEG_STEPS = 9
WIN_STEPS = 4
POS_STEPS = 4

_mesh = plsc.VectorSubcoreMesh(
    core_axis_name="c", subcore_axis_name="s", num_cores=NC, num_subcores=NS
)
_params = pltpu.CompilerParams(
    needs_layout_passes=False, use_tc_tiling_on_sc=False
)


def _wid():
    return lax.axis_index("s") * NC + lax.axis_index("c")


def _scan_body(x_ref, o_ref, w_ref, t_ref):
    x = x_ref[...]
    sh = 1
    while sh < SEG:
        left = jnp.zeros((NSEG, sh), jnp.float32)
        x = x + jnp.concatenate([left, x[:, : SEG - sh]], axis=1)
        sh *= 2
    o_ref[...] = x
    w_ref[...] = x.reshape(NSEG, WPS, L)[:, :, L - 1]
    t_ref[...] = x[:, SEG - 1]


_tc_scan = pl.pallas_call(
    _scan_body,
    out_shape=(
        jax.ShapeDtypeStruct((NSEG, SEG), jnp.float32),
        jax.ShapeDtypeStruct((NSEG, WPS), jnp.float32),
        jax.ShapeDtypeStruct((NSEG,), jnp.float32),
    ),
)


@functools.partial(
    pl.kernel,
    out_type=jax.ShapeDtypeStruct((NU,), jnp.int32),
    mesh=_mesh,
    compiler_params=_params,
    scratch_types=[
        pltpu.VMEM((NSEG,), jnp.float32),
        pltpu.VMEM((NSEG,), jnp.float32),
        pltpu.VMEM((NSEG,), jnp.float32),
        pltpu.VMEM((NWIN,), jnp.float32),
        pltpu.VMEM((UPW,), jnp.float32),
        pltpu.VMEM((2, UPW // 2), jnp.int32),
        pltpu.VMEM((2, UPW // 2, L), jnp.float32),
        pltpu.VMEM((UPW,), jnp.int32),
        pltpu.SemaphoreType.DMA,
        pltpu.SemaphoreType.DMA,
    ],
)
def _search(lcdfw_hbm, win_hbm, segtot_hbm, u_hbm, out_hbm,
            sb_v, gseg_v, gexc_v, win_v, u_v, widx_v, rows_v, out_v,
            sem, sem2):
    w = _wid()
    cps = (
        pltpu.async_copy(segtot_hbm, sb_v, sem),
        pltpu.async_copy(win_hbm, win_v, sem),
        pltpu.async_copy(u_hbm.at[pl.ds(w * UPW, UPW)], u_v, sem),
    )
    for cp in cps:
        cp.wait()

    carry = jnp.float32(0.0)
    for r in range(NSEG // L):
        v = sb_v[pl.ds(r * L, L)]
        cinc = plsc.cumsum(v) + carry
        gseg_v[pl.ds(r * L, L)] = cinc
        gexc_v[pl.ds(r * L, L)] = cinc - v
        carry = cinc[L - 1]
    total = carry

    ii = lax.iota(jnp.int32, L)
    HB = UVR // 2

    def _levels12(jset):
        ts = {j: u_v[pl.ds(j * L, L)] * total for j in jset}
        los = {j: jnp.zeros((L,), jnp.int32) for j in jset}
        his = {j: jnp.full((L,), NSEG - 1, jnp.int32) for j in jset}
        for _ in range(SEG_STEPS):
            for j in jset:
                mid = lax.shift_right_logical(los[j] + his[j], 1)
                g = plsc.load_gather(gseg_v, [mid])
                left = ts[j] <= g
                los[j] = jnp.where(left, los[j], mid + 1)
                his[j] = jnp.where(left, mid, his[j])
        tts, bases = {}, {}
        for j in jset:
            s = los[j]
            tts[j] = ts[j] - plsc.load_gather(gexc_v, [s])
            bases[j] = s * WPS
            los[j] = jnp.zeros((L,), jnp.int32)
            his[j] = jnp.full((L,), WPS - 1, jnp.int32)
        for _ in range(WIN_STEPS):
            for j in jset:
                mid = lax.shift_right_logical(los[j] + his[j], 1)
                g = plsc.load_gather(win_v, [bases[j] + mid])
                left = tts[j] <= g
                los[j] = jnp.where(left, los[j], mid + 1)
                his[j] = jnp.where(left, mid, his[j])
        widxs = {}
        for j in jset:
            widxs[j] = bases[j] + los[j]
            widx_v[j // HB, pl.ds((j % HB) * L, L)] = widxs[j]
        return tts, widxs

    def _level3(jset, tts, widxs, h):
        los = {j: jnp.zeros((L,), jnp.int32) for j in jset}
        his = {j: jnp.full((L,), L - 1, jnp.int32) for j in jset}
        hv = jnp.full((L,), h, jnp.int32)
        for _ in range(POS_STEPS):
            for j in jset:
                mid = lax.shift_right_logical(los[j] + his[j], 1)
                c = plsc.load_gather(
                    rows_v, [hv, ii + (j % HB) * L, mid]
                )
                left = tts[j] <= c
                los[j] = jnp.where(left, los[j], mid + 1)
                his[j] = jnp.where(left, mid, his[j])
        for j in jset:
            out_v[pl.ds(j * L, L)] = jnp.minimum(widxs[j] * L + los[j], N)

    set_a = range(0, HB)
    set_b = range(HB, UVR)
    tts_a, widxs_a = _levels12(set_a)
    cp_a = pltpu.async_copy(lcdfw_hbm.at[widx_v.at[0]], rows_v.at[0], sem)
    tts_b, widxs_b = _levels12(set_b)
    cp_b = pltpu.async_copy(lcdfw_hbm.at[widx_v.at[1]], rows_v.at[1], sem2)
    cp_a.wait()
    _level3(set_a, tts_a, widxs_a, 0)
    cp_b.wait()
    _level3(set_b, tts_b, widxs_b, 1)
    pltpu.sync_copy(out_v, out_hbm.at[pl.ds(w * UPW, UPW)])


def kernel(probs, x):
    dims = tuple(x.shape[:-1]) + (1,)
    n = 1
    for d in dims:
        n *= d
    assert n == NU and probs.shape == (N,)
    pp = jnp.concatenate(
        [probs.astype(jnp.float32), jnp.zeros((NP - N,), jnp.float32)]
    ).reshape(NSEG, SEG)
    u = jax.random.uniform(jax.random.key(42), (n,), dtype=jnp.float32)
    lcdf, win, segtot = _tc_scan(pp)
    samples = _search(lcdf.reshape(NWIN, L), win.reshape(NWIN), segtot, u)
    return samples.reshape(dims)


# speedup vs baseline: 1.0295x; 1.0280x over previous
import functools

import jax
import jax.numpy as jnp
from jax import lax
from jax.experimental import pallas as pl
from jax.experimental.pallas import tpu as pltpu
from jax.experimental.pallas import tpu_sc as plsc

N = 100000
NC, NS, L = 2, 16, 16
NW = NC * NS
SEG = 208
WPS = SEG // L
NSEG = NW * L
NP = NSEG * SEG
NWIN = NSEG * WPS
CHUNK = SEG * L
NU = 4096
UPW = NU // NW
UVR = UPW // L
SEG_STEPS = 9
WIN_STEPS = 4
POS_STEPS = 4

_mesh = plsc.VectorSubcoreMesh(
    core_axis_name="c", subcore_axis_name="s", num_cores=NC, num_subcores=NS
)
_params = pltpu.CompilerParams(
    needs_layout_passes=False, use_tc_tiling_on_sc=False
)


def _wid():
    return lax.axis_index("s") * NC + lax.axis_index("c")


def _scan_body(x_ref, o_ref, w_ref, t_ref):
    ri = lax.broadcasted_iota(jnp.int32, (SEG, SEG), 0)
    ci = lax.broadcasted_iota(jnp.int32, (SEG, SEG), 1)
    tri = jnp.where(ri <= ci, jnp.float32(1.0), jnp.float32(0.0))
    x = jnp.dot(x_ref[...], tri, preferred_element_type=jnp.float32)
    o_ref[...] = x
    w_ref[...] = x.reshape(NSEG, WPS, L)[:, :, L - 1]
    t_ref[...] = x[:, SEG - 1]


_tc_scan = pl.pallas_call(
    _scan_body,
    out_shape=(
        jax.ShapeDtypeStruct((NSEG, SEG), jnp.float32),
        jax.ShapeDtypeStruct((NSEG, WPS), jnp.float32),
        jax.ShapeDtypeStruct((NSEG,), jnp.float32),
    ),
)


@functools.partial(
    pl.kernel,
    out_type=jax.ShapeDtypeStruct((NU,), jnp.int32),
    mesh=_mesh,
    compiler_params=_params,
    scratch_types=[
        pltpu.VMEM((NSEG,), jnp.float32),
        pltpu.VMEM((NSEG,), jnp.float32),
        pltpu.VMEM((NSEG,), jnp.float32),
        pltpu.VMEM((NWIN,), jnp.float32),
        pltpu.VMEM((UPW,), jnp.float32),
        pltpu.VMEM((2, UPW // 2), jnp.int32),
        pltpu.VMEM((2, UPW // 2, L), jnp.float32),
        pltpu.VMEM((UPW,), jnp.int32),
        pltpu.SemaphoreType.DMA,
        pltpu.SemaphoreType.DMA,
    ],
)
def _search(lcdfw_hbm, win_hbm, segtot_hbm, u_hbm, out_hbm,
            sb_v, gseg_v, gexc_v, win_v, u_v, widx_v, rows_v, out_v,
            sem, sem2):
    w = _wid()
    cps = (
        pltpu.async_copy(segtot_hbm, sb_v, sem),
        pltpu.async_copy(win_hbm, win_v, sem),
        pltpu.async_copy(u_hbm.at[pl.ds(w * UPW, UPW)], u_v, sem),
    )
    for cp in cps:
        cp.wait()

    carry = jnp.float32(0.0)
    for r in range(NSEG // L):
        v = sb_v[pl.ds(r * L, L)]
        cinc = plsc.cumsum(v) + carry
        gseg_v[pl.ds(r * L, L)] = cinc
        gexc_v[pl.ds(r * L, L)] = cinc - v
        carry = cinc[L - 1]
    total = carry

    ii = lax.iota(jnp.int32, L)
    HB = UVR // 2

    def _levels12(jset):
        ts = {j: u_v[pl.ds(j * L, L)] * total for j in jset}
        los = {j: jnp.zeros((L,), jnp.int32) for j in jset}
        his = {j: jnp.full((L,), NSEG - 1, jnp.int32) for j in jset}
        for _ in range(SEG_STEPS):
            for j in jset:
                mid = lax.shift_right_logical(los[j] + his[j], 1)
                g = plsc.load_gather(gseg_v, [mid])
                left = ts[j] <= g
                los[j] = jnp.where(left, los[j], mid + 1)
                his[j] = jnp.where(left, mid, his[j])
        tts, bases = {}, {}
        for j in jset:
            s = los[j]
            tts[j] = ts[j] - plsc.load_gather(gexc_v, [s])
            bases[j] = s * WPS
            los[j] = jnp.zeros((L,), jnp.int32)
            his[j] = jnp.full((L,), WPS - 1, jnp.int32)
        for _ in range(WIN_STEPS):
            for j in jset:
                mid = lax.shift_right_logical(los[j] + his[j], 1)
                g = plsc.load_gather(win_v, [bases[j] + mid])
                left = tts[j] <= g
                los[j] = jnp.where(left, los[j], mid + 1)
                his[j] = jnp.where(left, mid, his[j])
        widxs = {}
        for j in jset:
            widxs[j] = bases[j] + los[j]
            widx_v[j // HB, pl.ds((j % HB) * L, L)] = widxs[j]
        return tts, widxs

    def _level3(jset, tts, widxs, h):
        los = {j: jnp.zeros((L,), jnp.int32) for j in jset}
        his = {j: jnp.full((L,), L - 1, jnp.int32) for j in jset}
        hv = jnp.full((L,), h, jnp.int32)
        for _ in range(POS_STEPS):
            for j in jset:
                mid = lax.shift_right_logical(los[j] + his[j], 1)
                c = plsc.load_gather(
                    rows_v, [hv, ii + (j % HB) * L, mid]
                )
                left = tts[j] <= c
                los[j] = jnp.where(left, los[j], mid + 1)
                his[j] = jnp.where(left, mid, his[j])
        for j in jset:
            out_v[pl.ds(j * L, L)] = jnp.minimum(widxs[j] * L + los[j], N)

    set_a = range(0, HB)
    set_b = range(HB, UVR)
    tts_a, widxs_a = _levels12(set_a)
    cp_a = pltpu.async_copy(lcdfw_hbm.at[widx_v.at[0]], rows_v.at[0], sem)
    tts_b, widxs_b = _levels12(set_b)
    cp_b = pltpu.async_copy(lcdfw_hbm.at[widx_v.at[1]], rows_v.at[1], sem2)
    cp_a.wait()
    _level3(set_a, tts_a, widxs_a, 0)
    cp_b.wait()
    _level3(set_b, tts_b, widxs_b, 1)
    pltpu.sync_copy(out_v, out_hbm.at[pl.ds(w * UPW, UPW)])


def kernel(probs, x):
    dims = tuple(x.shape[:-1]) + (1,)
    n = 1
    for d in dims:
        n *= d
    assert n == NU and probs.shape == (N,)
    pp = jnp.concatenate(
        [probs.astype(jnp.float32), jnp.zeros((NP - N,), jnp.float32)]
    ).reshape(NSEG, SEG)
    u = jax.random.uniform(jax.random.key(42), (n,), dtype=jnp.float32)
    lcdf, win, segtot = _tc_scan(pp)
    samples = _search(lcdf.reshape(NWIN, L), win.reshape(NWIN), segtot, u)
    return samples.reshape(dims)
